# baseline (speedup 1.0000x reference)
import functools

import jax
import jax.numpy as jnp
from jax import lax
from jax.experimental import pallas as pl
from jax.experimental.pallas import tpu as pltpu
from jax.experimental.pallas import tpu_sc as plsc

B = 16384
D = 64
V = 1000000
L = 16
NUM_CORES = 2
NUM_SUBCORES = 16
NW = NUM_CORES * NUM_SUBCORES
NBLK = (V + 127) // 128
BLK_PER_W = 245
W31_FULL = NBLK - 1 - 31 * BLK_PER_W
TAIL_W = V - (NBLK - 1) * 128
NBUF = 4
SROWS = 64
SM_CNT = 0
SM_START = 256
SM_CUR = 512
SM_NBL = 768
SM_KROW = 1020
SM_QN = 1021
SM_NNB = 1022


def _hsum(x):
    return jnp.sum(x, axis=0)


@jax.jit
def _dict_layer(ids, table_t):
    mesh = plsc.VectorSubcoreMesh(
        core_axis_name="c", subcore_axis_name="s",
        num_cores=NUM_CORES, num_subcores=NUM_SUBCORES)

    @functools.partial(
        pl.kernel,
        mesh=mesh,
        compiler_params=pltpu.CompilerParams(needs_layout_passes=False),
        out_type=jax.ShapeDtypeStruct((B + SROWS, 128), jnp.float32),
        scratch_types=[
            pltpu.VMEM((B + L,), jnp.int32),
            pltpu.VMEM((B + L,), jnp.int32),
            pltpu.VMEM((B + L,), jnp.int32),
            pltpu.VMEM((B + L,), jnp.int32),
            [pltpu.VMEM((D, 128), jnp.float32) for _ in range(NBUF)],
            pltpu.VMEM((D, TAIL_W), jnp.float32),
            pltpu.VMEM((SROWS, 128), jnp.float32),
            pltpu.VMEM((SROWS,), jnp.int32),
            pltpu.SMEM((1024,), jnp.int32),
            [pltpu.SemaphoreType.DMA for _ in range(NBUF)],
            pltpu.SemaphoreType.DMA,
        ],
    )
    def sc_kernel(ids_hbm, tab_hbm, out_hbm,
                  ids_v, mid_v, mpos_v, spos_v, blk_v, tail_v,
                  stage_v, stpos_v, sm, gsems, osem):
        wid = lax.axis_index("s") * NUM_CORES + lax.axis_index("c")
        is_last = wid == NW - 1
        b0 = wid * BLK_PER_W
        n_full = jnp.where(is_last, W31_FULL, BLK_PER_W)
        n_all = jnp.where(is_last, W31_FULL + 1, BLK_PER_W)
        iota = lax.iota(jnp.int32, L)
        scale = jnp.float32(0.1 / D)

        def fire_direct(slot, b_rel):
            gb = b0 + b_rel
            off = pl.multiple_of(gb * 128, 128)
            return pltpu.async_copy(
                tab_hbm.at[:, pl.ds(off, 128)], blk_v[slot], gsems[slot])

        for s in range(NBUF):
            fire_direct(s, s)

        pltpu.sync_copy(ids_hbm, ids_v.at[pl.ds(0, B)])

        def zero_body(i, c):
            sm[i] = 0
            return c
        lax.fori_loop(0, 768, zero_body, 0)

        def match_body(g, qn):
            idv = ids_v[pl.ds(g * L, L)]
            blk = lax.shift_right_logical(idv, 7)
            m = (blk >= b0) & (blk < b0 + n_all)
            cnt = plsc.all_reduce_population_count(m)[0]
            plsc.store_compressed(mid_v.at[pl.ds(qn, L)], idv, mask=m)
            plsc.store_compressed(mpos_v.at[pl.ds(qn, L)], g * L + iota,
                                  mask=m)
            return qn + cnt
        qn = lax.fori_loop(0, B // L, match_body, jnp.int32(0))
        sm[SM_QN] = qn

        def count_body(qi, c):
            e = plsc.load_gather(mid_v, [jnp.full((L,), qi, jnp.int32)])[0]
            br = lax.shift_right_logical(e, 7) - b0
            sm[SM_CNT + br] = sm[SM_CNT + br] + 1
            return c
        lax.fori_loop(0, qn, count_body, 0)

        def prefix_body(b, carry):
            run, nnb = carry
            c = sm[SM_CNT + b]
            sm[SM_START + b] = run
            sm[SM_CUR + b] = run
            take = ((c > 0) | (b < NBUF)) & (b < n_full)
            @pl.when(take)
            def _():
                sm[SM_NBL + nnb] = b
            nnb = jnp.where(take, nnb + 1, nnb)
            return (run + c, nnb)
        _, nnb = lax.fori_loop(0, n_all, prefix_body,
                               (jnp.int32(0), jnp.int32(0)))
        sm[SM_NNB] = nnb

        def place_body(qi, c):
            qsplat = jnp.full((L,), qi, jnp.int32)
            e = plsc.load_gather(mid_v, [qsplat])[0]
            p = plsc.load_gather(mpos_v, [qsplat])[0]
            br = lax.shift_right_logical(e, 7) - b0
            w = sm[SM_CUR + br]
            sm[SM_CUR + br] = w + 1
            wsplat = jnp.full((L,), w, jnp.int32)
            lane0 = iota == 0
            plsc.store_scatter(ids_v, [wsplat],
                               jnp.full((L,), e & 127, jnp.int32), mask=lane0)
            plsc.store_scatter(spos_v, [wsplat],
                               jnp.full((L,), p, jnp.int32), mask=lane0)
            return c
        lax.fori_loop(0, qn, place_body, 0)

        def reset_stpos():
            for k in range(SROWS // L):
                stpos_v[pl.ds(k * L, L)] = B + k * L + iota
        reset_stpos()
        sm[SM_KROW] = 0

        def flush_stage():
            pltpu.async_copy(stage_v, out_hbm.at[stpos_v], osem).wait()
            reset_stpos()
            sm[SM_KROW] = 0

        def process_segment(qi, blk_ref, c):
            qsplat = jnp.full((L,), qi, jnp.int32)
            col = plsc.load_gather(ids_v, [qsplat])[0]
            p = plsc.load_gather(spos_v, [qsplat])[0]
            csplat = jnp.full((L,), col, jnp.int32)
            k = sm[SM_KROW]
            ksplat = jnp.full((L,), k, jnp.int32)
            acc = jnp.zeros((L,), jnp.float32)
            for f in range(D // L):
                feats = plsc.load_gather(blk_ref, [f * L + iota, csplat])
                plsc.store_scatter(stage_v, [ksplat, f * L + iota], feats)
                acc = acc + jnp.abs(feats)
            tot = _hsum(acc) * scale
            lane0 = iota == 0
            plsc.store_scatter(stage_v, [ksplat, jnp.full((L,), D, jnp.int32)],
                               jnp.full((L,), tot, jnp.float32), mask=lane0)
            plsc.store_scatter(stpos_v, [ksplat],
                               jnp.full((L,), p, jnp.int32), mask=lane0)
            sm[SM_KROW] = k + 1

            @pl.when(k + 1 == SROWS)
            def _():
                flush_stage()
            return c

        def fire(slot, j):
            gb = b0 + sm[SM_NBL + j]
            off = pl.multiple_of(gb * 128, 128)
            return pltpu.async_copy(
                tab_hbm.at[:, pl.ds(off, 128)], blk_v[slot], gsems[slot])

        n_outer = (BLK_PER_W + NBUF - 1) // NBUF
        def outer_body(ob, c):
            for s in range(NBUF):
                j = ob * NBUF + s

                @pl.when(j < sm[SM_NNB])
                def _(s=s, j=j):
                    pltpu.make_async_copy(
                        tab_hbm.at[:, pl.ds(0, 128)], blk_v[s],
                        gsems[s]).wait()
                    b_rel = sm[SM_NBL + j]
                    seg0 = sm[SM_START + b_rel]
                    seg1 = seg0 + sm[SM_CNT + b_rel]
                    lax.fori_loop(
                        seg0, seg1,
                        lambda qi, cc: process_segment(qi, blk_v[s], cc), 0)

                    @pl.when(j + NBUF < sm[SM_NNB])
                    def _():
                        fire(s, j + NBUF)
            return c
        lax.fori_loop(0, n_outer, outer_body, 0)

        @pl.when(is_last)
        def _():
            pltpu.sync_copy(
                tab_hbm.at[:, pl.ds((NBLK - 1) * 128, TAIL_W)], tail_v)
            seg0 = sm[SM_START + W31_FULL]
            seg1 = seg0 + sm[SM_CNT + W31_FULL]
            lax.fori_loop(
                seg0, seg1,
                lambda qi, cc: process_segment(qi, tail_v, cc), 0)

        flush_stage()

    out2 = sc_kernel(ids, table_t)

    @functools.partial(
        pl.kernel,
        mesh=mesh,
        compiler_params=pltpu.CompilerParams(needs_layout_passes=False),
        out_type=(jax.ShapeDtypeStruct((D, B), jnp.float32),
                  jax.ShapeDtypeStruct((B,), jnp.float32)),
        scratch_types=[
            pltpu.VMEM((128, 128), jnp.float32),
            pltpu.VMEM((D, 128), jnp.float32),
            pltpu.VMEM((B // NW,), jnp.float32),
        ],
    )
    def unpack_kernel(out2_hbm, outt_hbm, loss_hbm, rows_v, outt_v, loss_v):
        wid = lax.axis_index("s") * NUM_CORES + lax.axis_index("c")
        bpw = B // NW
        base = wid * bpw
        iota = lax.iota(jnp.int32, L)
        for c in range(bpw // 128):
            off = base + c * 128
            pltpu.sync_copy(out2_hbm.at[pl.ds(off, 128)], rows_v)
            for g in range(128 // L):
                lv = plsc.load_gather(
                    rows_v, [g * L + iota, jnp.full((L,), D, jnp.int32)])
                loss_v[pl.ds(c * 128 + g * L, L)] = lv
            for f in range(D):
                fsplat = jnp.full((L,), f, jnp.int32)
                for j in range(128 // L):
                    outt_v[f, pl.ds(j * L, L)] = plsc.load_gather(
                        rows_v, [j * L + iota, fsplat])
            pltpu.sync_copy(outt_v, outt_hbm.at[:, pl.ds(off, 128)])
        pltpu.sync_copy(loss_v, loss_hbm.at[pl.ds(base, bpw)])

    return unpack_kernel(out2)


def kernel(input, kernel):
    ids = input.astype(jnp.int32)
    outt, loss1 = _dict_layer(ids, kernel.T)
    fetched = outt.T
    loss = loss1.reshape(B, 1)
    return (fetched, loss)


# speedup vs baseline: 1.1498x; 1.1498x over previous
import functools

import jax
import jax.numpy as jnp
from jax import lax
from jax.experimental import pallas as pl
from jax.experimental.pallas import tpu as pltpu
from jax.experimental.pallas import tpu_sc as plsc

B = 16384
D = 64
V = 1000000
L = 16
NUM_CORES = 2
NUM_SUBCORES = 16
NW = NUM_CORES * NUM_SUBCORES
NBLK = (V + 127) // 128
BLK_PER_W = 245
W31_FULL = NBLK - 1 - 31 * BLK_PER_W
TAIL_W = V - (NBLK - 1) * 128
NBUF = 4
SROWS = 64
SM_CNT = 0
SM_START = 256
SM_CUR = 512
SM_NBL = 768
SM_KROW = 1020
SM_QN = 1021
SM_NNB = 1022


def _hsum(x):
    return jnp.sum(x, axis=0)


@jax.jit
def _dict_layer(ids, table_t):
    mesh = plsc.VectorSubcoreMesh(
        core_axis_name="c", subcore_axis_name="s",
        num_cores=NUM_CORES, num_subcores=NUM_SUBCORES)

    @functools.partial(
        pl.kernel,
        mesh=mesh,
        compiler_params=pltpu.CompilerParams(needs_layout_passes=False),
        out_type=jax.ShapeDtypeStruct((B + SROWS, 128), jnp.float32),
        scratch_types=[
            pltpu.VMEM((B + L,), jnp.int32),
            pltpu.VMEM((B + L,), jnp.int32),
            pltpu.VMEM((B + L,), jnp.int32),
            pltpu.VMEM((B + L,), jnp.int32),
            [pltpu.VMEM((D, 128), jnp.float32) for _ in range(NBUF)],
            pltpu.VMEM((D, TAIL_W), jnp.float32),
            pltpu.VMEM((SROWS, 128), jnp.float32),
            pltpu.VMEM((SROWS,), jnp.int32),
            pltpu.SMEM((1024,), jnp.int32),
            [pltpu.SemaphoreType.DMA for _ in range(NBUF)],
            pltpu.SemaphoreType.DMA,
        ],
    )
    def sc_kernel(ids_hbm, tab_hbm, out_hbm,
                  ids_v, mid_v, mpos_v, spos_v, blk_v, tail_v,
                  stage_v, stpos_v, sm, gsems, osem):
        wid = lax.axis_index("s") * NUM_CORES + lax.axis_index("c")
        is_last = wid == NW - 1
        b0 = wid * BLK_PER_W
        n_full = jnp.where(is_last, W31_FULL, BLK_PER_W)
        n_all = jnp.where(is_last, W31_FULL + 1, BLK_PER_W)
        iota = lax.iota(jnp.int32, L)
        scale = jnp.float32(0.1 / D)

        def fire_direct(slot, b_rel):
            gb = b0 + b_rel
            off = pl.multiple_of(gb * 128, 128)
            return pltpu.async_copy(
                tab_hbm.at[:, pl.ds(off, 128)], blk_v[slot], gsems[slot])

        for s in range(NBUF):
            fire_direct(s, s)

        pltpu.sync_copy(ids_hbm, ids_v.at[pl.ds(0, B)])

        def zero_body(i, c):
            sm[i] = 0
            return c
        lax.fori_loop(0, 768, zero_body, 0)

        def match_body(g, qn):
            idv = ids_v[pl.ds(g * L, L)]
            blk = lax.shift_right_logical(idv, 7)
            m = (blk >= b0) & (blk < b0 + n_all)
            cnt = plsc.all_reduce_population_count(m)[0]
            plsc.store_compressed(mid_v.at[pl.ds(qn, L)], idv, mask=m)
            plsc.store_compressed(mpos_v.at[pl.ds(qn, L)], g * L + iota,
                                  mask=m)
            return qn + cnt
        qn = lax.fori_loop(0, B // L, match_body, jnp.int32(0))
        sm[SM_QN] = qn

        def count_body(qi, c):
            e = plsc.load_gather(mid_v, [jnp.full((L,), qi, jnp.int32)])[0]
            br = lax.shift_right_logical(e, 7) - b0
            sm[SM_CNT + br] = sm[SM_CNT + br] + 1
            return c
        lax.fori_loop(0, qn, count_body, 0)

        def prefix_body(b, carry):
            run, nnb = carry
            c = sm[SM_CNT + b]
            sm[SM_START + b] = run
            sm[SM_CUR + b] = run
            take = ((c > 0) | (b < NBUF)) & (b < n_full)
            @pl.when(take)
            def _():
                sm[SM_NBL + nnb] = b
            nnb = jnp.where(take, nnb + 1, nnb)
            return (run + c, nnb)
        _, nnb = lax.fori_loop(0, n_all, prefix_body,
                               (jnp.int32(0), jnp.int32(0)))
        sm[SM_NNB] = nnb

        def place_body(qi, c):
            qsplat = jnp.full((L,), qi, jnp.int32)
            e = plsc.load_gather(mid_v, [qsplat])[0]
            p = plsc.load_gather(mpos_v, [qsplat])[0]
            br = lax.shift_right_logical(e, 7) - b0
            w = sm[SM_CUR + br]
            sm[SM_CUR + br] = w + 1
            wsplat = jnp.full((L,), w, jnp.int32)
            lane0 = iota == 0
            plsc.store_scatter(ids_v, [wsplat],
                               jnp.full((L,), e & 127, jnp.int32), mask=lane0)
            plsc.store_scatter(spos_v, [wsplat],
                               jnp.full((L,), p, jnp.int32), mask=lane0)
            return c
        lax.fori_loop(0, qn, place_body, 0)

        def reset_stpos():
            for k in range(SROWS // L):
                stpos_v[pl.ds(k * L, L)] = B + k * L + iota
        reset_stpos()
        sm[SM_KROW] = 0

        def flush_stage():
            pltpu.async_copy(stage_v, out_hbm.at[stpos_v], osem).wait()
            reset_stpos()
            sm[SM_KROW] = 0

        def process_segment(qi, blk_ref, c):
            qsplat = jnp.full((L,), qi, jnp.int32)
            col = plsc.load_gather(ids_v, [qsplat])[0]
            p = plsc.load_gather(spos_v, [qsplat])[0]
            csplat = jnp.full((L,), col, jnp.int32)
            k = sm[SM_KROW]
            ksplat = jnp.full((L,), k, jnp.int32)
            acc = jnp.zeros((L,), jnp.float32)
            for f in range(D // L):
                feats = plsc.load_gather(blk_ref, [f * L + iota, csplat])
                plsc.store_scatter(stage_v, [ksplat, f * L + iota], feats)
                acc = acc + jnp.abs(feats)
            tot = _hsum(acc) * scale
            lane0 = iota == 0
            plsc.store_scatter(stage_v, [ksplat, jnp.full((L,), D, jnp.int32)],
                               jnp.full((L,), tot, jnp.float32), mask=lane0)
            plsc.store_scatter(stpos_v, [ksplat],
                               jnp.full((L,), p, jnp.int32), mask=lane0)
            sm[SM_KROW] = k + 1

            @pl.when(k + 1 == SROWS)
            def _():
                flush_stage()
            return c

        def fire(slot, j):
            gb = b0 + sm[SM_NBL + j]
            off = pl.multiple_of(gb * 128, 128)
            return pltpu.async_copy(
                tab_hbm.at[:, pl.ds(off, 128)], blk_v[slot], gsems[slot])

        n_outer = (BLK_PER_W + NBUF - 1) // NBUF
        def outer_body(ob, c):
            for s in range(NBUF):
                j = ob * NBUF + s

                @pl.when(j < sm[SM_NNB])
                def _(s=s, j=j):
                    pltpu.make_async_copy(
                        tab_hbm.at[:, pl.ds(0, 128)], blk_v[s],
                        gsems[s]).wait()
                    b_rel = sm[SM_NBL + j]
                    seg0 = sm[SM_START + b_rel]
                    seg1 = seg0 + sm[SM_CNT + b_rel]
                    lax.fori_loop(
                        seg0, seg1,
                        lambda qi, cc: process_segment(qi, blk_v[s], cc), 0)

                    @pl.when(j + NBUF < sm[SM_NNB])
                    def _():
                        fire(s, j + NBUF)
            return c
        lax.fori_loop(0, n_outer, outer_body, 0)

        @pl.when(is_last)
        def _():
            pltpu.sync_copy(
                tab_hbm.at[:, pl.ds((NBLK - 1) * 128, TAIL_W)], tail_v)
            seg0 = sm[SM_START + W31_FULL]
            seg1 = seg0 + sm[SM_CNT + W31_FULL]
            lax.fori_loop(
                seg0, seg1,
                lambda qi, cc: process_segment(qi, tail_v, cc), 0)

        flush_stage()

    return sc_kernel(ids, table_t)


def kernel(input, kernel):
    ids = input.astype(jnp.int32)
    out2 = _dict_layer(ids, kernel.T)
    fetched = out2[:B, :D]
    loss = out2[:B, D:D + 1]
    return (fetched, loss)
